# baseline (speedup 1.0000x reference)
import functools

import jax
import jax.numpy as jnp
from jax import lax
from jax.experimental import pallas as pl
from jax.experimental.pallas import tpu as pltpu
from jax.experimental.pallas import tpu_sc as plsc

BATCH = 16384
HIST = 50
DIM = 16
TOTAL = BATCH * HIST

NC = 2
NS = 16
NW = NC * NS
B_PER_W = TOTAL // NW
CHUNK = 3200
NCHUNK = B_PER_W // CHUNK


NSTREAM = 4
SUB = CHUNK // NSTREAM


def _split_gather(table_hbm, idx_v, rows_v, sem):
    return [
        pltpu.async_copy(
            table_hbm.at[idx_v.at[pl.ds(k * SUB, SUB)]],
            rows_v.at[pl.ds(k * SUB, SUB)], sem)
        for k in range(NSTREAM)
    ]


@functools.partial(
    pl.kernel,
    out_type=jax.ShapeDtypeStruct((TOTAL, DIM), jnp.float32),
    mesh=plsc.VectorSubcoreMesh(core_axis_name="c", subcore_axis_name="s"),
    scratch_types=[
        pltpu.VMEM((CHUNK,), jnp.int32),
        pltpu.VMEM((CHUNK,), jnp.int32),
        pltpu.VMEM((CHUNK, DIM), jnp.float32),
        pltpu.VMEM((CHUNK, DIM), jnp.float32),
        pltpu.SemaphoreType.DMA,
        pltpu.SemaphoreType.DMA,
        pltpu.SemaphoreType.DMA,
        pltpu.SemaphoreType.DMA,
        pltpu.SemaphoreType.DMA,
        pltpu.SemaphoreType.DMA,
    ],
    compiler_params=pltpu.CompilerParams(use_tc_tiling_on_sc=False),
)
def _gather(idx_hbm, table_hbm, out_hbm, i0, i1, r0, r1,
            si0, si1, sg0, sg1, so0, so1):
    wid = lax.axis_index("s") * NC + lax.axis_index("c")
    base = wid * B_PER_W
    idx_v = (i0, i1)
    rows_v = (r0, r1)
    sg = (sg0, sg1)
    so = (so0, so1)

    def off(i):
        return base + i * CHUNK

    idx_cp = [
        pltpu.async_copy(idx_hbm.at[pl.ds(off(0), CHUNK)], i0, si0),
        pltpu.async_copy(idx_hbm.at[pl.ds(off(1), CHUNK)], i1, si1),
    ]
    gat_cp = [None, None]
    out_cp = [None, None]

    idx_cp[0].wait()
    gat_cp[0] = _split_gather(table_hbm, i0, r0, sg[0])

    for i in range(NCHUNK):
        b = i % 2
        nb = (i + 1) % 2
        if i + 1 < NCHUNK:
            idx_cp[nb].wait()
            if out_cp[nb] is not None:
                out_cp[nb].wait()
            gat_cp[nb] = _split_gather(table_hbm, idx_v[nb], rows_v[nb],
                                       sg[nb])
        for cp in gat_cp[b]:
            cp.wait()
        out_cp[b] = pltpu.async_copy(
            rows_v[b], out_hbm.at[pl.ds(off(i), CHUNK)], so[b])
        if i + 2 < NCHUNK:
            idx_cp[b] = pltpu.async_copy(
                idx_hbm.at[pl.ds(off(i + 2), CHUNK)], idx_v[b],
                (si0, si1)[b])

    out_cp[0].wait()
    out_cp[1].wait()


def kernel(action_history, embedding_weight):
    idx = action_history.reshape(-1).astype(jnp.int32)
    out = _gather(idx, embedding_weight)
    return out.reshape(action_history.shape[0], -1)


# speedup vs baseline: 1.0239x; 1.0239x over previous
import functools

import jax
import jax.numpy as jnp
from jax import lax
from jax.experimental import pallas as pl
from jax.experimental.pallas import tpu as pltpu
from jax.experimental.pallas import tpu_sc as plsc

BATCH = 16384
HIST = 50
DIM = 16
TOTAL = BATCH * HIST
NROWS = 100000

NC = 2
NS = 16
NW = NC * NS
B_PER_W = TOTAL // NW
CHUNK = 800
NCHUNK = B_PER_W // CHUNK
ROWS_PER_TILE = NROWS // NS


@functools.partial(
    pl.kernel,
    out_type=jax.ShapeDtypeStruct((TOTAL, DIM), jnp.float32),
    mesh=plsc.VectorSubcoreMesh(core_axis_name="c", subcore_axis_name="s"),
    scratch_types=[
        pltpu.VMEM_SHARED((NROWS, DIM), jnp.float32),
        pltpu.VMEM((CHUNK,), jnp.int32),
        pltpu.VMEM((CHUNK,), jnp.int32),
        pltpu.VMEM((CHUNK, DIM), jnp.float32),
        pltpu.VMEM((CHUNK, DIM), jnp.float32),
        pltpu.SemaphoreType.DMA,
        pltpu.SemaphoreType.DMA,
        pltpu.SemaphoreType.DMA,
        pltpu.SemaphoreType.DMA,
        pltpu.SemaphoreType.DMA,
        pltpu.SemaphoreType.DMA,
    ],
    compiler_params=pltpu.CompilerParams(use_tc_tiling_on_sc=False),
)
def _gather(idx_hbm, table_hbm, out_hbm, tbl_sh, i0, i1, r0, r1,
            si0, si1, sg0, sg1, so0, so1):
    sid = lax.axis_index("s")
    wid = sid * NC + lax.axis_index("c")
    base = wid * B_PER_W
    idx_v = (i0, i1)
    rows_v = (r0, r1)
    sg = (sg0, sg1)
    so = (so0, so1)

    def off(i):
        return base + i * CHUNK

    trow = sid * ROWS_PER_TILE
    stage = pltpu.async_copy(
        table_hbm.at[pl.ds(trow, ROWS_PER_TILE)],
        tbl_sh.at[pl.ds(trow, ROWS_PER_TILE)], sg0)
    idx_cp = [
        pltpu.async_copy(idx_hbm.at[pl.ds(off(0), CHUNK)], i0, si0),
        pltpu.async_copy(idx_hbm.at[pl.ds(off(1), CHUNK)], i1, si1),
    ]
    stage.wait()
    plsc.subcore_barrier()

    gat_cp = [None, None]
    out_cp = [None, None]

    idx_cp[0].wait()
    gat_cp[0] = pltpu.async_copy(tbl_sh.at[i0], r0, sg[0])

    for i in range(NCHUNK):
        b = i % 2
        nb = (i + 1) % 2
        if i + 1 < NCHUNK:
            idx_cp[nb].wait()
            if out_cp[nb] is not None:
                out_cp[nb].wait()
            gat_cp[nb] = pltpu.async_copy(
                tbl_sh.at[idx_v[nb]], rows_v[nb], sg[nb])
        gat_cp[b].wait()
        out_cp[b] = pltpu.async_copy(
            rows_v[b], out_hbm.at[pl.ds(off(i), CHUNK)], so[b])
        if i + 2 < NCHUNK:
            idx_cp[b] = pltpu.async_copy(
                idx_hbm.at[pl.ds(off(i + 2), CHUNK)], idx_v[b],
                (si0, si1)[b])

    out_cp[0].wait()
    out_cp[1].wait()


def kernel(action_history, embedding_weight):
    idx = action_history.reshape(-1).astype(jnp.int32)
    out = _gather(idx, embedding_weight)
    return out.reshape(action_history.shape[0], -1)
